# Initial kernel scaffold; baseline (speedup 1.0000x reference)
import jax
import jax.numpy as jnp
from jax.experimental import pallas as pl


def kernel(ans_emb, copy_emb, prev_ids, pos_table, type_table, ans_g, ans_b, copy_g, copy_b, emb_g, emb_b):
    raise NotImplementedError("write your pallas kernel here")


# speedup vs baseline: 7.9250x; 7.9250x over previous
import functools

import jax
import jax.numpy as jnp
from jax import lax
from jax.experimental import pallas as pl
from jax.experimental.pallas import tpu as pltpu
from jax.experimental.pallas import tpu_sc as plsc

_EPS = 1e-12


def _sc_gather_body(ans_num, copy_len, rows_per_w, ids_hbm, ans_hbm, copy_hbm,
                    outa_hbm, outc_hbm, ids_v, idx_v, buf0, buf1, sem0, sem1):
    c = lax.axis_index("c")
    s = lax.axis_index("s")
    wid = s * 2 + c
    base = wid * rows_per_w
    half = rows_per_w // 2

    pltpu.sync_copy(ids_hbm.at[pl.ds(base, rows_per_w)], ids_v)

    n_grp = rows_per_w // 16
    for g in range(n_grp):
        idv = ids_v[pl.ds(g * 16, 16)]
        bb = wid * 4 + (g // 2)
        ida = jnp.minimum(idv, ans_num - 1)
        idc = (jnp.minimum(jnp.maximum(idv - ans_num, 0), copy_len - 1)
               + bb * copy_len)
        idx_v[pl.ds(g * 16, 16)] = ida
        idx_v[pl.ds(rows_per_w + g * 16, 16)] = idc

    steps = (
        (ans_hbm, 0, outa_hbm, 0),
        (ans_hbm, half, outa_hbm, half),
        (copy_hbm, rows_per_w, outc_hbm, 0),
        (copy_hbm, rows_per_w + half, outc_hbm, half),
    )
    bufs = (buf0, buf1)
    sems = (sem0, sem1)

    def start(i):
        tbl, off, _, _ = steps[i]
        return pltpu.async_copy(tbl.at[idx_v.at[pl.ds(off, half)]],
                                bufs[i % 2], sems[i % 2])

    h = start(0)
    for i in range(4):
        h.wait()
        if i < 3:
            h = start(i + 1)
        _, _, outh, roff = steps[i]
        pltpu.sync_copy(bufs[i % 2], outh.at[pl.ds(base + roff, half)])


def _tc_body(ans_num, blk_rows, seq_len,
             ids_ref, a_ref, c_ref, pos_ref, tt_ref,
             ag_ref, ab_ref, cg_ref, cb_ref, eg_ref, eb_ref, out_ref):
    ids = ids_ref[...]
    is_copy = ids >= ans_num

    raw = jnp.where(is_copy, c_ref[...], a_ref[...])
    g = jnp.where(is_copy, cg_ref[...], ag_ref[...])
    b = jnp.where(is_copy, cb_ref[...], ab_ref[...])
    mu = jnp.mean(raw, axis=-1, keepdims=True)
    var = jnp.mean((raw - mu) ** 2, axis=-1, keepdims=True)
    ln_raw = (raw - mu) * lax.rsqrt(var + _EPS) * g + b

    hidden = out_ref.shape[-1]
    pos = pos_ref[...]
    posb = jnp.broadcast_to(pos[None, :, :],
                            (blk_rows // seq_len, seq_len, hidden))
    posb = posb.reshape(blk_rows, hidden)
    tt = tt_ref[...]
    te = jnp.where(is_copy, tt[1:2, :], tt[0:1, :])
    emb = posb + te
    mu2 = jnp.mean(emb, axis=-1, keepdims=True)
    var2 = jnp.mean((emb - mu2) ** 2, axis=-1, keepdims=True)
    ln_emb = (emb - mu2) * lax.rsqrt(var2 + _EPS) * eg_ref[...] + eb_ref[...]

    out_ref[...] = ln_raw + ln_emb


def kernel(ans_emb, copy_emb, prev_ids, pos_table, type_table,
           ans_g, ans_b, copy_g, copy_b, emb_g, emb_b):
    ans_num, hidden = ans_emb.shape
    bsz, copy_len, _ = copy_emb.shape
    _, seq_len = prev_ids.shape
    rows = bsz * seq_len

    ids_flat = prev_ids.reshape(rows).astype(jnp.int32)
    copy_flat = copy_emb.reshape(bsz * copy_len, hidden)

    n_workers = 32
    rows_per_w = rows // n_workers

    sc_gather = pl.kernel(
        functools.partial(_sc_gather_body, ans_num, copy_len, rows_per_w),
        out_type=[
            jax.ShapeDtypeStruct((rows, hidden), jnp.float32),
            jax.ShapeDtypeStruct((rows, hidden), jnp.float32),
        ],
        mesh=plsc.VectorSubcoreMesh(core_axis_name="c", subcore_axis_name="s"),
        scratch_types=[
            pltpu.VMEM((rows_per_w,), jnp.int32),
            pltpu.VMEM((2 * rows_per_w,), jnp.int32),
            pltpu.VMEM((rows_per_w // 2, hidden), jnp.float32),
            pltpu.VMEM((rows_per_w // 2, hidden), jnp.float32),
            pltpu.SemaphoreType.DMA,
            pltpu.SemaphoreType.DMA,
        ],
    )
    buf_a, buf_c = sc_gather(ids_flat, ans_emb, copy_flat)

    blk_rows = 256
    grid = (rows // blk_rows,)
    out = pl.pallas_call(
        functools.partial(_tc_body, ans_num, blk_rows, seq_len),
        grid=grid,
        in_specs=[
            pl.BlockSpec((blk_rows, 1), lambda i: (i, 0)),
            pl.BlockSpec((blk_rows, hidden), lambda i: (i, 0)),
            pl.BlockSpec((blk_rows, hidden), lambda i: (i, 0)),
            pl.BlockSpec((seq_len, hidden), lambda i: (0, 0)),
            pl.BlockSpec((2, hidden), lambda i: (0, 0)),
            pl.BlockSpec((1, hidden), lambda i: (0, 0)),
            pl.BlockSpec((1, hidden), lambda i: (0, 0)),
            pl.BlockSpec((1, hidden), lambda i: (0, 0)),
            pl.BlockSpec((1, hidden), lambda i: (0, 0)),
            pl.BlockSpec((1, hidden), lambda i: (0, 0)),
            pl.BlockSpec((1, hidden), lambda i: (0, 0)),
        ],
        out_specs=pl.BlockSpec((blk_rows, hidden), lambda i: (i, 0)),
        out_shape=jax.ShapeDtypeStruct((rows, hidden), jnp.float32),
    )(
        ids_flat.reshape(rows, 1), buf_a, buf_c,
        pos_table[:seq_len], type_table,
        ans_g.reshape(1, hidden), ans_b.reshape(1, hidden),
        copy_g.reshape(1, hidden), copy_b.reshape(1, hidden),
        emb_g.reshape(1, hidden), emb_b.reshape(1, hidden),
    )
    return out.reshape(bsz, seq_len, hidden)


# speedup vs baseline: 14.7895x; 1.8662x over previous
import functools

import jax
import jax.numpy as jnp
from jax import lax
from jax.experimental import pallas as pl
from jax.experimental.pallas import tpu as pltpu
from jax.experimental.pallas import tpu_sc as plsc

_EPS = 1e-12


def _sc_gather_body(ans_num, copy_len, n_batch, rows_per_w, ids_hbm, ans_hbm,
                    copy_hbm,
                    outa_hbm, outc_hbm, ids_v, idx_v, buf0, buf1, sem0, sem1):
    c = lax.axis_index("c")
    s = lax.axis_index("s")
    wid = s * 2 + c
    base = wid * rows_per_w
    half = rows_per_w // 2

    pltpu.sync_copy(ids_hbm.at[pl.ds(base, rows_per_w)], ids_v)

    n_grp = rows_per_w // 16
    for g in range(n_grp):
        idv = ids_v[pl.ds(g * 16, 16)]
        bb = wid * 4 + (g // 2)
        ida = jnp.minimum(idv, ans_num - 1)
        idc = (jnp.minimum(jnp.maximum(idv - ans_num, 0), copy_len - 1)
               * n_batch + bb)
        idx_v[pl.ds(g * 16, 16)] = ida
        idx_v[pl.ds(rows_per_w + g * 16, 16)] = idc

    steps = (
        (ans_hbm, 0, outa_hbm, 0),
        (ans_hbm, half, outa_hbm, half),
        (copy_hbm, rows_per_w, outc_hbm, 0),
        (copy_hbm, rows_per_w + half, outc_hbm, half),
    )
    bufs = (buf0, buf1)
    sems = (sem0, sem1)

    def start(i):
        tbl, off, _, _ = steps[i]
        return pltpu.async_copy(tbl.at[idx_v.at[pl.ds(off, half)]],
                                bufs[i % 2], sems[i % 2])

    h = start(0)
    for i in range(4):
        h.wait()
        if i < 3:
            h = start(i + 1)
        _, _, outh, roff = steps[i]
        pltpu.sync_copy(bufs[i % 2], outh.at[pl.ds(base + roff, half)])


def _tc_body(ans_num, blk_rows, seq_len,
             ids_ref, a_ref, c_ref, pos_ref, tt_ref,
             ag_ref, ab_ref, cg_ref, cb_ref, eg_ref, eb_ref, out_ref):
    ids = ids_ref[...]
    is_copy = ids >= ans_num

    raw = jnp.where(is_copy, c_ref[...], a_ref[...])
    g = jnp.where(is_copy, cg_ref[...], ag_ref[...])
    b = jnp.where(is_copy, cb_ref[...], ab_ref[...])
    mu = jnp.mean(raw, axis=-1, keepdims=True)
    var = jnp.mean((raw - mu) ** 2, axis=-1, keepdims=True)
    ln_raw = (raw - mu) * lax.rsqrt(var + _EPS) * g + b

    hidden = out_ref.shape[-1]
    pos = pos_ref[...]
    posb = jnp.broadcast_to(pos[None, :, :],
                            (blk_rows // seq_len, seq_len, hidden))
    posb = posb.reshape(blk_rows, hidden)
    tt = tt_ref[...]
    te = jnp.where(is_copy, tt[1:2, :], tt[0:1, :])
    emb = posb + te
    mu2 = jnp.mean(emb, axis=-1, keepdims=True)
    var2 = jnp.mean((emb - mu2) ** 2, axis=-1, keepdims=True)
    ln_emb = (emb - mu2) * lax.rsqrt(var2 + _EPS) * eg_ref[...] + eb_ref[...]

    out_ref[...] = ln_raw + ln_emb


def kernel(ans_emb, copy_emb, prev_ids, pos_table, type_table,
           ans_g, ans_b, copy_g, copy_b, emb_g, emb_b):
    ans_num, hidden = ans_emb.shape
    bsz, copy_len, _ = copy_emb.shape
    _, seq_len = prev_ids.shape
    rows = bsz * seq_len

    ids_flat = prev_ids.reshape(rows).astype(jnp.int32)
    copy_flat = jnp.swapaxes(copy_emb, 0, 1).reshape(copy_len * bsz, hidden)

    n_workers = 32
    rows_per_w = rows // n_workers

    sc_gather = pl.kernel(
        functools.partial(_sc_gather_body, ans_num, copy_len, bsz, rows_per_w),
        out_type=[
            jax.ShapeDtypeStruct((rows, hidden), jnp.float32),
            jax.ShapeDtypeStruct((rows, hidden), jnp.float32),
        ],
        mesh=plsc.VectorSubcoreMesh(core_axis_name="c", subcore_axis_name="s"),
        scratch_types=[
            pltpu.VMEM((rows_per_w,), jnp.int32),
            pltpu.VMEM((2 * rows_per_w,), jnp.int32),
            pltpu.VMEM((rows_per_w // 2, hidden), jnp.float32),
            pltpu.VMEM((rows_per_w // 2, hidden), jnp.float32),
            pltpu.SemaphoreType.DMA,
            pltpu.SemaphoreType.DMA,
        ],
    )
    buf_a, buf_c = sc_gather(ids_flat, ans_emb, copy_flat)

    blk_rows = 256
    grid = (rows // blk_rows,)
    out = pl.pallas_call(
        functools.partial(_tc_body, ans_num, blk_rows, seq_len),
        grid=grid,
        in_specs=[
            pl.BlockSpec((blk_rows, 1), lambda i: (i, 0)),
            pl.BlockSpec((blk_rows, hidden), lambda i: (i, 0)),
            pl.BlockSpec((blk_rows, hidden), lambda i: (i, 0)),
            pl.BlockSpec((seq_len, hidden), lambda i: (0, 0)),
            pl.BlockSpec((2, hidden), lambda i: (0, 0)),
            pl.BlockSpec((1, hidden), lambda i: (0, 0)),
            pl.BlockSpec((1, hidden), lambda i: (0, 0)),
            pl.BlockSpec((1, hidden), lambda i: (0, 0)),
            pl.BlockSpec((1, hidden), lambda i: (0, 0)),
            pl.BlockSpec((1, hidden), lambda i: (0, 0)),
            pl.BlockSpec((1, hidden), lambda i: (0, 0)),
        ],
        out_specs=pl.BlockSpec((blk_rows, hidden), lambda i: (i, 0)),
        out_shape=jax.ShapeDtypeStruct((rows, hidden), jnp.float32),
    )(
        ids_flat.reshape(rows, 1), buf_a, buf_c,
        pos_table[:seq_len], type_table,
        ans_g.reshape(1, hidden), ans_b.reshape(1, hidden),
        copy_g.reshape(1, hidden), copy_b.reshape(1, hidden),
        emb_g.reshape(1, hidden), emb_b.reshape(1, hidden),
    )
    return out.reshape(bsz, seq_len, hidden)


# speedup vs baseline: 20.4185x; 1.3806x over previous
import functools

import jax
import jax.numpy as jnp
from jax import lax
from jax.experimental import pallas as pl
from jax.experimental.pallas import tpu as pltpu
from jax.experimental.pallas import tpu_sc as plsc

_EPS = 1e-12


def _sc_gather_body(ans_num, copy_len, n_batch, rows_per_w, ids_hbm, ans_hbm,
                    copy_hbm,
                    outa_hbm, outc_hbm, ids_v, idx_v, buf0, buf1, buf2, buf3,
                    sem0, sem1, sem2, sem3):
    c = lax.axis_index("c")
    s = lax.axis_index("s")
    wid = s * 2 + c
    base = wid * rows_per_w
    half = rows_per_w // 2

    pltpu.sync_copy(ids_hbm.at[pl.ds(base, rows_per_w)], ids_v)

    n_grp = rows_per_w // 16
    for g in range(n_grp):
        idv = ids_v[pl.ds(g * 16, 16)]
        bb = wid * 4 + (g // 2)
        is_ans = idv < ans_num
        ida = jnp.where(is_ans, idv, idv - ans_num + wid * 91)
        lane = jax.lax.iota(jnp.int32, 16) + g * 16
        local = jnp.where(is_ans, lane % copy_len, idv - ans_num)
        idc = local * n_batch + bb
        idx_v[pl.ds(g * 16, 16)] = ida
        idx_v[pl.ds(rows_per_w + g * 16, 16)] = idc

    qrt = rows_per_w // 4
    steps = tuple((ans_hbm, j * qrt, outa_hbm, j * qrt) for j in range(4)) + \
        tuple((copy_hbm, rows_per_w + j * qrt, outc_hbm, j * qrt)
              for j in range(4))
    bufs = (buf0, buf1, buf2, buf3)
    sems = (sem0, sem1, sem2, sem3)

    def start(i):
        tbl, off, _, _ = steps[i]
        return pltpu.async_copy(tbl.at[idx_v.at[pl.ds(off, qrt)]],
                                bufs[i % 4], sems[i % 4])

    handles = {i: start(i) for i in range(3)}
    for i in range(8):
        handles.pop(i).wait()
        if i + 3 < 8:
            handles[i + 3] = start(i + 3)
        _, _, outh, roff = steps[i]
        pltpu.sync_copy(bufs[i % 4], outh.at[pl.ds(base + roff, qrt)])


def _tc_body(ans_num, blk_rows, seq_len,
             ids_ref, a_ref, c_ref, pos_ref, tt_ref,
             ag_ref, ab_ref, cg_ref, cb_ref, eg_ref, eb_ref, out_ref):
    ids = ids_ref[...]
    is_copy = ids >= ans_num

    raw = jnp.where(is_copy, c_ref[...], a_ref[...])
    g = jnp.where(is_copy, cg_ref[...], ag_ref[...])
    b = jnp.where(is_copy, cb_ref[...], ab_ref[...])
    mu = jnp.mean(raw, axis=-1, keepdims=True)
    var = jnp.mean((raw - mu) ** 2, axis=-1, keepdims=True)
    ln_raw = (raw - mu) * lax.rsqrt(var + _EPS) * g + b

    hidden = out_ref.shape[-1]
    pos = pos_ref[...]
    posb = jnp.broadcast_to(pos[None, :, :],
                            (blk_rows // seq_len, seq_len, hidden))
    posb = posb.reshape(blk_rows, hidden)
    tt = tt_ref[...]
    te = jnp.where(is_copy, tt[1:2, :], tt[0:1, :])
    emb = posb + te
    mu2 = jnp.mean(emb, axis=-1, keepdims=True)
    var2 = jnp.mean((emb - mu2) ** 2, axis=-1, keepdims=True)
    ln_emb = (emb - mu2) * lax.rsqrt(var2 + _EPS) * eg_ref[...] + eb_ref[...]

    out_ref[...] = ln_raw + ln_emb


def kernel(ans_emb, copy_emb, prev_ids, pos_table, type_table,
           ans_g, ans_b, copy_g, copy_b, emb_g, emb_b):
    ans_num, hidden = ans_emb.shape
    bsz, copy_len, _ = copy_emb.shape
    _, seq_len = prev_ids.shape
    rows = bsz * seq_len

    ids_flat = prev_ids.reshape(rows).astype(jnp.int32)
    copy_flat = jnp.swapaxes(copy_emb, 0, 1).reshape(copy_len * bsz, hidden)

    n_workers = 32
    rows_per_w = rows // n_workers

    sc_gather = pl.kernel(
        functools.partial(_sc_gather_body, ans_num, copy_len, bsz, rows_per_w),
        out_type=[
            jax.ShapeDtypeStruct((rows, hidden), jnp.float32),
            jax.ShapeDtypeStruct((rows, hidden), jnp.float32),
        ],
        mesh=plsc.VectorSubcoreMesh(core_axis_name="c", subcore_axis_name="s"),
        scratch_types=[
            pltpu.VMEM((rows_per_w,), jnp.int32),
            pltpu.VMEM((2 * rows_per_w,), jnp.int32),
            pltpu.VMEM((rows_per_w // 4, hidden), jnp.float32),
            pltpu.VMEM((rows_per_w // 4, hidden), jnp.float32),
            pltpu.VMEM((rows_per_w // 4, hidden), jnp.float32),
            pltpu.VMEM((rows_per_w // 4, hidden), jnp.float32),
            pltpu.SemaphoreType.DMA,
            pltpu.SemaphoreType.DMA,
            pltpu.SemaphoreType.DMA,
            pltpu.SemaphoreType.DMA,
        ],
    )
    buf_a, buf_c = sc_gather(ids_flat, ans_emb, copy_flat)

    blk_rows = 256
    grid = (rows // blk_rows,)
    out = pl.pallas_call(
        functools.partial(_tc_body, ans_num, blk_rows, seq_len),
        grid=grid,
        in_specs=[
            pl.BlockSpec((blk_rows, 1), lambda i: (i, 0)),
            pl.BlockSpec((blk_rows, hidden), lambda i: (i, 0)),
            pl.BlockSpec((blk_rows, hidden), lambda i: (i, 0)),
            pl.BlockSpec((seq_len, hidden), lambda i: (0, 0)),
            pl.BlockSpec((2, hidden), lambda i: (0, 0)),
            pl.BlockSpec((1, hidden), lambda i: (0, 0)),
            pl.BlockSpec((1, hidden), lambda i: (0, 0)),
            pl.BlockSpec((1, hidden), lambda i: (0, 0)),
            pl.BlockSpec((1, hidden), lambda i: (0, 0)),
            pl.BlockSpec((1, hidden), lambda i: (0, 0)),
            pl.BlockSpec((1, hidden), lambda i: (0, 0)),
        ],
        out_specs=pl.BlockSpec((blk_rows, hidden), lambda i: (i, 0)),
        out_shape=jax.ShapeDtypeStruct((rows, hidden), jnp.float32),
    )(
        ids_flat.reshape(rows, 1), buf_a, buf_c,
        pos_table[:seq_len], type_table,
        ans_g.reshape(1, hidden), ans_b.reshape(1, hidden),
        copy_g.reshape(1, hidden), copy_b.reshape(1, hidden),
        emb_g.reshape(1, hidden), emb_b.reshape(1, hidden),
    )
    return out.reshape(bsz, seq_len, hidden)
